# baseline (speedup 1.0000x reference)
import jax
import jax.numpy as jnp
from jax import lax
from jax.experimental import pallas as pl
from jax.experimental.pallas import tpu as pltpu
from jax.experimental.pallas import tpu_sc as plsc

_DIMS = [119, 5, 12, 12, 10, 6, 6, 2, 2]
_K = 8
_D = 248
_DOUT = 256
_N = 100000
_XC = 17

_NC = 2
_NS = 16
_NW = _NC * _NS
_CHUNK = 96
_NCH = (_N + _CHUNK - 1) // _CHUNK
_NP = _NCH * _CHUNK
_TAILROWS = _N - (_NCH - 1) * _CHUNK
_TITER = 33
_TAILW = (_NCH - 1) % _NW
_XCH = _XC * _CHUNK
_OCH = _CHUNK * _DOUT

_PAIRS = [(2, 8), (3, 7), (4, 1), (5, 6)]
_SECROWS = [_DIMS[0]] + [_DIMS[a] * _DIMS[b] for a, b in _PAIRS]
_SSTART = [0]
for _r in _SECROWS[:-1]:
    _SSTART.append(_SSTART[-1] + _r)
_TROWS = sum(_SECROWS)


def _compute_chunk(x_v, t_v, out_v, iota):
    def group_body(g, carry):
        r0 = g * 16
        riota = iota + r0
        rowbase = riota * _DOUT

        ci = [
            x_v[pl.ds((_K + i) * _CHUNK + r0, 16)].astype(jnp.int32)
            for i in range(9)
        ]
        addrs8 = [ci[0] * _DOUT - _K]
        for p, (a, b) in enumerate(_PAIRS):
            addrs8.append(
                ci[a] * (_DIMS[b] * _DOUT)
                + ci[b] * _DOUT
                + (_SSTART[1 + p] * _DOUT - _K)
            )

        @plsc.parallel_loop(0, 16, unroll=4)
        def _(u):
            colv = (iota + u) & 15
            m = colv >= _K
            tcs = jnp.maximum(colv, _K)
            tv = None
            for k in range(5):
                gk = plsc.load_gather(t_v, [addrs8[k] + tcs])
                tv = gk if tv is None else tv + gk
            xcol = jnp.minimum(colv, _XC - 1)
            cv = plsc.load_gather(x_v, [xcol * _CHUNK + riota])
            plsc.store_scatter(out_v, [rowbase + colv], jnp.where(m, tv, cv))

        @plsc.parallel_loop(16, _DOUT, unroll=8)
        def _(cc):
            colv = ((iota + cc) & 15) + (cc & ~15)
            v = None
            for k in range(5):
                gk = plsc.load_gather(t_v, [addrs8[k] + colv])
                v = gk if v is None else v + gk
            plsc.store_scatter(out_v, [rowbase + colv], v)

        return carry

    lax.fori_loop(0, 6, group_body, 0)


def _sc_body(x_hbm, t_hbm, out_hbm, xv0, xv1, ov0, ov1, t_v, sx0, sx1, so0, so1):
    wid = lax.axis_index("s") * _NC + lax.axis_index("c")
    pltpu.sync_copy(t_hbm, t_v)
    iota = lax.iota(jnp.int32, 16)
    xbufs = [xv0, xv1]
    obufs = [ov0, ov1]
    sxs = [sx0, sx1]
    sos = [so0, so1]

    def cid_of(t):
        return wid + t * _NW

    def x_src(t):
        return x_hbm.at[pl.ds(cid_of(t) * _XCH, _XCH)]

    pltpu.async_copy(x_src(0), xv0, sx0)

    @pl.loop(0, _TITER, step=2)
    def _(tt):
        for b in range(2):
            t = tt + b

            @pl.when((t < _TITER) & (cid_of(t) < _NCH))
            def _():
                cid = cid_of(t)
                pltpu.make_async_copy(x_src(t), xbufs[b], sxs[b]).wait()

                @pl.when((t + 1 < _TITER) & (cid + _NW < _NCH))
                def _():
                    pltpu.async_copy(x_src(t + 1), xbufs[1 - b], sxs[1 - b])

                @pl.when(t >= 2)
                def _():
                    pltpu.make_async_copy(
                        obufs[b], out_hbm.at[pl.ds(0, _OCH)], sos[b]
                    ).wait()

                _compute_chunk(xbufs[b], t_v, obufs[b], iota)

                @pl.when(cid < _NCH - 1)
                def _():
                    pltpu.async_copy(
                        obufs[b], out_hbm.at[pl.ds(cid * _OCH, _OCH)], sos[b]
                    )

                @pl.when(cid == _NCH - 1)
                def _():
                    pltpu.async_copy(
                        obufs[b].at[pl.ds(0, _TAILROWS * _DOUT)],
                        out_hbm.at[pl.ds(cid * _OCH, _TAILROWS * _DOUT)],
                        sos[b],
                    )

    pltpu.make_async_copy(ov1, out_hbm.at[pl.ds(0, _OCH)], so1).wait()

    @pl.when(wid == _TAILW)
    def _():
        pltpu.make_async_copy(
            ov0.at[pl.ds(0, _TAILROWS * _DOUT)],
            out_hbm.at[pl.ds(0, _TAILROWS * _DOUT)],
            so0,
        ).wait()

    @pl.when(wid != _TAILW)
    def _():
        pltpu.make_async_copy(ov0, out_hbm.at[pl.ds(0, _OCH)], so0).wait()


@jax.jit
def _run(xt_flat, t_flat):
    mesh = plsc.VectorSubcoreMesh(core_axis_name="c", subcore_axis_name="s")
    f = pl.kernel(
        _sc_body,
        mesh=mesh,
        compiler_params=pltpu.CompilerParams(
            needs_layout_passes=False, use_tc_tiling_on_sc=False
        ),
        out_type=jax.ShapeDtypeStruct((_N * _DOUT,), jnp.float32),
        scratch_types=[
            pltpu.VMEM((_XCH,), jnp.float32),
            pltpu.VMEM((_XCH,), jnp.float32),
            pltpu.VMEM((_OCH,), jnp.float32),
            pltpu.VMEM((_OCH,), jnp.float32),
            pltpu.VMEM((_TROWS * _DOUT,), jnp.float32),
            pltpu.SemaphoreType.DMA,
            pltpu.SemaphoreType.DMA,
            pltpu.SemaphoreType.DMA,
            pltpu.SemaphoreType.DMA,
        ],
    )
    return f(xt_flat, t_flat)


def kernel(x, W0, W1, W2, W3, W4, W5, W6, W7, W8):
    Ws = [W0, W1, W2, W3, W4, W5, W6, W7, W8]
    secs = [W0]
    for a, b in _PAIRS:
        secs.append((Ws[a][:, None, :] + Ws[b][None, :, :]).reshape(-1, _D))
    table = jnp.concatenate(secs, axis=0)
    table = jnp.pad(table, ((0, 0), (0, _DOUT - _D)))
    xp = jnp.pad(x, ((0, _NP - _N), (0, 0)))
    xt = xp.reshape(_NCH, _CHUNK, _XC).transpose(0, 2, 1)
    out = _run(xt.reshape(-1), table.reshape(-1))
    return out.reshape(_N, _DOUT)


# speedup vs baseline: 1.0857x; 1.0857x over previous
import jax
import jax.numpy as jnp
from jax import lax
from jax.experimental import pallas as pl
from jax.experimental.pallas import tpu as pltpu
from jax.experimental.pallas import tpu_sc as plsc

_DIMS = [119, 5, 12, 12, 10, 6, 6, 2, 2]
_K = 8
_D = 248
_DOUT = 256
_N = 100000
_XC = 17

_NC = 2
_NS = 16
_NW = _NC * _NS
_CHUNK = 32
_NCH = (_N + _CHUNK - 1) // _CHUNK
_NP = _NCH * _CHUNK
_TAILROWS = _N - (_NCH - 1) * _CHUNK
_TITER = (_NCH + _NW - 1) // _NW
_TAILW = (_NCH - 1) % _NW
_XCH = _XC * _CHUNK
_OCH = _CHUNK * _DOUT

_SECROWS = [_DIMS[0], _DIMS[2] * _DIMS[4], _DIMS[3] * _DIMS[5],
            _DIMS[1] * _DIMS[6] * _DIMS[7] * _DIMS[8]]
_SSTART = [0]
for _r in _SECROWS[:-1]:
    _SSTART.append(_SSTART[-1] + _r)
_TROWS = sum(_SECROWS)
_NTAB = 4


def _compute_chunk(x_v, t_v, out_v, iota):
    def group_body(g, carry):
        r0 = g * 16
        riota = iota + r0
        rowbase = riota * _DOUT

        ci = [
            x_v[pl.ds((_K + i) * _CHUNK + r0, 16)].astype(jnp.int32)
            for i in range(9)
        ]
        addrs8 = [
            ci[0] * _DOUT - _K,
            (ci[2] * _DIMS[4] + ci[4] + _SSTART[1]) * _DOUT - _K,
            (ci[3] * _DIMS[5] + ci[5] + _SSTART[2]) * _DOUT - _K,
            (
                ci[1] * (_DIMS[6] * _DIMS[7] * _DIMS[8])
                + ci[6] * (_DIMS[7] * _DIMS[8])
                + ci[7] * _DIMS[8]
                + ci[8]
                + _SSTART[3]
            )
            * _DOUT
            - _K,
        ]

        @plsc.parallel_loop(0, 16, unroll=4)
        def _(u):
            colv = (iota + u) & 15
            m = colv >= _K
            tcs = jnp.maximum(colv, _K)
            tv = None
            for k in range(_NTAB):
                gk = plsc.load_gather(t_v, [addrs8[k] + tcs])
                tv = gk if tv is None else tv + gk
            xcol = jnp.minimum(colv, _XC - 1)
            cv = plsc.load_gather(x_v, [xcol * _CHUNK + riota])
            plsc.store_scatter(out_v, [rowbase + colv], jnp.where(m, tv, cv))

        @plsc.parallel_loop(16, _DOUT, unroll=8)
        def _(cc):
            colv = ((iota + cc) & 15) + (cc & ~15)
            v = None
            for k in range(_NTAB):
                gk = plsc.load_gather(t_v, [addrs8[k] + colv])
                v = gk if v is None else v + gk
            plsc.store_scatter(out_v, [rowbase + colv], v)

        return carry

    lax.fori_loop(0, _CHUNK // 16, group_body, 0)


def _sc_body(x_hbm, t_hbm, out_hbm, xv0, xv1, ov0, ov1, t_v, sx0, sx1, so0, so1):
    wid = lax.axis_index("s") * _NC + lax.axis_index("c")
    pltpu.sync_copy(t_hbm, t_v)
    iota = lax.iota(jnp.int32, 16)
    xbufs = [xv0, xv1]
    obufs = [ov0, ov1]
    sxs = [sx0, sx1]
    sos = [so0, so1]

    def cid_of(t):
        return wid + t * _NW

    def x_src(t):
        return x_hbm.at[pl.ds(cid_of(t) * _XCH, _XCH)]

    pltpu.async_copy(x_src(0), xv0, sx0)

    @pl.loop(0, _TITER, step=2)
    def _(tt):
        for b in range(2):
            t = tt + b

            @pl.when((t < _TITER) & (cid_of(t) < _NCH))
            def _():
                cid = cid_of(t)
                pltpu.make_async_copy(x_src(t), xbufs[b], sxs[b]).wait()

                @pl.when((t + 1 < _TITER) & (cid + _NW < _NCH))
                def _():
                    pltpu.async_copy(x_src(t + 1), xbufs[1 - b], sxs[1 - b])

                @pl.when(t >= 2)
                def _():
                    pltpu.make_async_copy(
                        obufs[b], out_hbm.at[pl.ds(0, _OCH)], sos[b]
                    ).wait()

                _compute_chunk(xbufs[b], t_v, obufs[b], iota)

                @pl.when(cid < _NCH - 1)
                def _():
                    pltpu.async_copy(
                        obufs[b], out_hbm.at[pl.ds(cid * _OCH, _OCH)], sos[b]
                    )

                @pl.when(cid == _NCH - 1)
                def _():
                    pltpu.async_copy(
                        obufs[b].at[pl.ds(0, _TAILROWS * _DOUT)],
                        out_hbm.at[pl.ds(cid * _OCH, _TAILROWS * _DOUT)],
                        sos[b],
                    )

    pltpu.make_async_copy(ov1, out_hbm.at[pl.ds(0, _OCH)], so1).wait()

    @pl.when(wid == _TAILW)
    def _():
        pltpu.make_async_copy(
            ov0.at[pl.ds(0, _TAILROWS * _DOUT)],
            out_hbm.at[pl.ds(0, _TAILROWS * _DOUT)],
            so0,
        ).wait()

    @pl.when(wid != _TAILW)
    def _():
        pltpu.make_async_copy(ov0, out_hbm.at[pl.ds(0, _OCH)], so0).wait()


@jax.jit
def _run(xt_flat, t_flat):
    mesh = plsc.VectorSubcoreMesh(core_axis_name="c", subcore_axis_name="s")
    f = pl.kernel(
        _sc_body,
        mesh=mesh,
        compiler_params=pltpu.CompilerParams(
            needs_layout_passes=False, use_tc_tiling_on_sc=False
        ),
        out_type=jax.ShapeDtypeStruct((_N * _DOUT,), jnp.float32),
        scratch_types=[
            pltpu.VMEM((_XCH,), jnp.float32),
            pltpu.VMEM((_XCH,), jnp.float32),
            pltpu.VMEM((_OCH,), jnp.float32),
            pltpu.VMEM((_OCH,), jnp.float32),
            pltpu.VMEM((_TROWS * _DOUT,), jnp.float32),
            pltpu.SemaphoreType.DMA,
            pltpu.SemaphoreType.DMA,
            pltpu.SemaphoreType.DMA,
            pltpu.SemaphoreType.DMA,
        ],
    )
    return f(xt_flat, t_flat)


def kernel(x, W0, W1, W2, W3, W4, W5, W6, W7, W8):
    s24 = (W2[:, None, :] + W4[None, :, :]).reshape(-1, _D)
    s35 = (W3[:, None, :] + W5[None, :, :]).reshape(-1, _D)
    s1678 = (
        W1[:, None, None, None, :]
        + W6[None, :, None, None, :]
        + W7[None, None, :, None, :]
        + W8[None, None, None, :, :]
    ).reshape(-1, _D)
    table = jnp.concatenate([W0, s24, s35, s1678], axis=0)
    table = jnp.pad(table, ((0, 0), (0, _DOUT - _D)))
    xp = jnp.pad(x, ((0, _NP - _N), (0, 0)))
    xt = xp.reshape(_NCH, _CHUNK, _XC).transpose(0, 2, 1)
    out = _run(xt.reshape(-1), table.reshape(-1))
    return out.reshape(_N, _DOUT)
